# baseline (speedup 1.0000x reference)
import functools

import jax
import jax.numpy as jnp
from jax import lax
from jax.experimental import pallas as pl
from jax.experimental.pallas import tpu as pltpu
from jax.experimental.pallas import tpu_sc as plsc


def kernel(sequence, table):
    batch = sequence.shape[0]
    seq_len = sequence.shape[2]
    vocab, dim = table.shape

    mesh = plsc.VectorSubcoreMesh(core_axis_name="c", subcore_axis_name="s")
    num_workers = mesh.num_cores * mesh.num_subcores

    assert seq_len % num_workers == 0
    rows_per_worker = seq_len // num_workers
    chunk = min(32, rows_per_worker)
    assert rows_per_worker % chunk == 0
    steps = rows_per_worker // chunk

    @functools.partial(
        pl.kernel,
        out_type=jax.ShapeDtypeStruct((batch, seq_len, dim), table.dtype),
        mesh=mesh,
        scratch_types=[
            pltpu.VMEM((2, chunk, dim), table.dtype),
            pltpu.SemaphoreType.DMA,
            pltpu.SemaphoreType.DMA,
        ],
    )
    def body(table_hbm, out_hbm, buf, gsem, wsem):
        wid = lax.axis_index("s") * mesh.num_cores + lax.axis_index("c")
        row0 = wid * rows_per_worker

        def gather(step):
            return pltpu.async_copy(
                table_hbm.at[pl.ds(row0 + step * chunk, chunk)],
                buf.at[step % 2],
                gsem,
            )

        writes = [None] * steps
        pending_gather = gather(0)
        for step in range(steps):
            cur = step % 2
            pending_gather.wait()
            if step >= 1:
                for h in writes[step - 1]:
                    h.wait()
            if step + 1 < steps:
                pending_gather = gather(step + 1)
            base = row0 + step * chunk
            writes[step] = [
                pltpu.async_copy(buf.at[cur], out_hbm.at[b, pl.ds(base, chunk)], wsem)
                for b in range(batch)
            ]
        for h in writes[steps - 1]:
            h.wait()

    return body(table)
